# baseline (speedup 1.0000x reference)
import jax
import jax.numpy as jnp
from jax import lax
from jax.experimental import pallas as pl
from jax.experimental.pallas import tpu as pltpu
from jax.experimental.pallas import tpu_sc as plsc

_NC = 2
_NS = 16
_NW = _NC * _NS
_L = 16

_NBUF = 4


def _sc_add_pe(x4, colors_flat, spat_flat, chrom_flat, w, hh, d):
    nb = x4.shape[0]
    n_lines = nb * hh
    half = d // 2
    vecs = half // _L
    lines_per_w = n_lines // _NW
    cols_per_w = lines_per_w * w

    mesh = plsc.VectorSubcoreMesh(core_axis_name="c", subcore_axis_name="s")

    @pl.kernel(
        out_type=jax.ShapeDtypeStruct((nb, hh, w, d), jnp.float32),
        mesh=mesh,
        compiler_params=pltpu.CompilerParams(
            needs_layout_passes=False, use_tc_tiling_on_sc=True),
        scratch_types=[
            pltpu.VMEM((hh * w * half,), jnp.float32),
            pltpu.VMEM((16 * half,), jnp.float32),
            pltpu.VMEM((cols_per_w + 16,), jnp.int32),
            pltpu.VMEM((w, d), jnp.float32),
            pltpu.VMEM((w, d), jnp.float32),
            pltpu.VMEM((w, d), jnp.float32),
            pltpu.VMEM((w, d), jnp.float32),
            pltpu.SemaphoreType.DMA,
            pltpu.SemaphoreType.DMA,
            pltpu.SemaphoreType.DMA,
            pltpu.SemaphoreType.DMA,
            pltpu.SemaphoreType.DMA,
            pltpu.SemaphoreType.DMA,
            pltpu.SemaphoreType.DMA,
            pltpu.SemaphoreType.DMA,
        ],
    )
    def body(x_hbm, col_hbm, spat_hbm, chrom_hbm, out_hbm,
             spat_v, chrom_v, col_v, rb0, rb1, rb2, rb3,
             si0, si1, si2, si3, so0, so1, so2, so3):
        wid = lax.axis_index("s") * _NC + lax.axis_index("c")
        base = wid * lines_per_w
        pltpu.sync_copy(spat_hbm, spat_v)
        pltpu.sync_copy(chrom_hbm, chrom_v.at[pl.ds(0, chrom_hbm.shape[0])])
        pltpu.sync_copy(col_hbm.at[pl.ds(wid * cols_per_w, cols_per_w)],
                        col_v.at[pl.ds(0, cols_per_w)])

        bufs = (rb0, rb1, rb2, rb3)
        isems = (si0, si1, si2, si3)
        osems = (so0, so1, so2, so3)

        def start_in(c, b):
            line = base + c
            pltpu.async_copy(x_hbm.at[lax.div(line, hh), lax.rem(line, hh)],
                             bufs[b], isems[b])

        def start_out(c, b):
            line = base + c
            pltpu.async_copy(bufs[b],
                             out_hbm.at[lax.div(line, hh), lax.rem(line, hh)],
                             osems[b])

        def wait_in(b):
            pltpu.make_async_copy(x_hbm.at[0, 0], bufs[b], isems[b]).wait()

        def wait_out(b):
            pltpu.make_async_copy(bufs[b], out_hbm.at[0, 0], osems[b]).wait()

        def compute(c, b):
            buf = bufs[b]
            line = base + c
            hbase = lax.rem(line, hh) * (w * half)
            coff = c * w
            col16a = col_v[pl.ds(coff, _L)]
            col16b = col_v[pl.ds(coff + _L, _L)]
            cbases = []
            for pw in range(w):
                crow = col16a[pw] if pw < _L else col16b[pw - _L]
                cbases.append(crow * half)
            for pw0 in range(0, w, 2):
                vals = []
                for pw in (pw0, pw0 + 1):
                    sbase = hbase + pw * half
                    cbase = cbases[pw]
                    vals += [spat_v[pl.ds(sbase + v * _L, _L)]
                             for v in range(vecs)]
                    vals += [chrom_v[pl.ds(cbase + v * _L, _L)]
                             for v in range(vecs)]
                for i, pw in enumerate((pw0, pw0 + 1)):
                    for v in range(vecs):
                        plsc.addupdate(buf.at[pw, pl.ds(v * _L, _L)],
                                       vals[2 * vecs * i + v])
                        plsc.addupdate(buf.at[pw, pl.ds(half + v * _L, _L)],
                                       vals[2 * vecs * i + vecs + v])

        start_in(0, 0)
        start_in(1, 1)

        def ring_body(g, carry):
            for b in range(_NBUF):
                c = g * _NBUF + b
                wait_in(b)
                compute(c, b)
                start_out(c, b)

                @pl.when(c >= 2)
                def _():
                    wait_out((b - 2) % _NBUF)

                @pl.when(c + 2 < lines_per_w)
                def _():
                    start_in(c + 2, (b + 2) % _NBUF)
            return carry

        lax.fori_loop(0, lines_per_w // _NBUF, ring_body, 0, unroll=False)
        wait_out((lines_per_w - 2) % _NBUF)
        wait_out((lines_per_w - 1) % _NBUF)

    return body(x4, colors_flat, spat_flat, chrom_flat)


def kernel(x, color_indices, spatial_pe, chromatic_pe):
    b, h, w, d = x.shape
    half = d // 2
    colors_flat = color_indices.reshape(b * h * w).astype(jnp.int32)
    spat_flat = spatial_pe[:h, :w, :].reshape(h * w * half)
    chrom_flat = chromatic_pe.reshape(-1)
    return _sc_add_pe(x, colors_flat, spat_flat, chrom_flat, w, h, d)


# speedup vs baseline: 1.0022x; 1.0022x over previous
import jax
import jax.numpy as jnp
from jax import lax
from jax.experimental import pallas as pl
from jax.experimental.pallas import tpu as pltpu
from jax.experimental.pallas import tpu_sc as plsc

_NC = 2
_NS = 16
_NW = _NC * _NS
_L = 16

_NBUF = 4


def _sc_add_pe(x4, colors_flat, spat_flat, chrom_flat, w, hh, d):
    nb = x4.shape[0]
    n_lines = nb * hh
    half = d // 2
    vecs = half // _L
    lines_per_w = n_lines // _NW
    cols_per_w = lines_per_w * w

    mesh = plsc.VectorSubcoreMesh(core_axis_name="c", subcore_axis_name="s")

    @pl.kernel(
        out_type=jax.ShapeDtypeStruct((nb, hh, w, d), jnp.float32),
        mesh=mesh,
        compiler_params=pltpu.CompilerParams(
            needs_layout_passes=False, use_tc_tiling_on_sc=True),
        scratch_types=[
            pltpu.VMEM((hh * w * half,), jnp.float32),
            pltpu.VMEM((16 * half,), jnp.float32),
            pltpu.VMEM((cols_per_w + 16,), jnp.int32),
            pltpu.VMEM((w, d), jnp.float32),
            pltpu.VMEM((w, d), jnp.float32),
            pltpu.VMEM((w, d), jnp.float32),
            pltpu.VMEM((w, d), jnp.float32),
            pltpu.SemaphoreType.DMA,
            pltpu.SemaphoreType.DMA,
            pltpu.SemaphoreType.DMA,
            pltpu.SemaphoreType.DMA,
            pltpu.SemaphoreType.DMA,
            pltpu.SemaphoreType.DMA,
            pltpu.SemaphoreType.DMA,
            pltpu.SemaphoreType.DMA,
        ],
    )
    def body(x_hbm, col_hbm, spat_hbm, chrom_hbm, out_hbm,
             spat_v, chrom_v, col_v, rb0, rb1, rb2, rb3,
             si0, si1, si2, si3, so0, so1, so2, so3):
        wid = lax.axis_index("s") * _NC + lax.axis_index("c")
        base = wid * lines_per_w
        pltpu.sync_copy(spat_hbm, spat_v)
        pltpu.sync_copy(chrom_hbm, chrom_v.at[pl.ds(0, chrom_hbm.shape[0])])
        pltpu.sync_copy(col_hbm.at[pl.ds(wid * cols_per_w, cols_per_w)],
                        col_v.at[pl.ds(0, cols_per_w)])

        bufs = (rb0, rb1, rb2, rb3)
        isems = (si0, si1, si2, si3)
        osems = (so0, so1, so2, so3)

        def start_in(c, b):
            line = base + c
            pltpu.async_copy(x_hbm.at[lax.div(line, hh), lax.rem(line, hh)],
                             bufs[b], isems[b])

        def start_out(c, b):
            line = base + c
            pltpu.async_copy(bufs[b],
                             out_hbm.at[lax.div(line, hh), lax.rem(line, hh)],
                             osems[b])

        def wait_in(b):
            pltpu.make_async_copy(x_hbm.at[0, 0], bufs[b], isems[b]).wait()

        def wait_out(b):
            pltpu.make_async_copy(bufs[b], out_hbm.at[0, 0], osems[b]).wait()

        def compute(c, b):
            buf = bufs[b]
            line = base + c
            hbase = lax.rem(line, hh) * (w * half)
            coff = c * w
            col16a = col_v[pl.ds(coff, _L)]
            col16b = col_v[pl.ds(coff + _L, _L)]
            cbases = []
            for pw in range(w):
                crow = col16a[pw] if pw < _L else col16b[pw - _L]
                cbases.append(crow * half)
            def loads(pw):
                sbase = hbase + pw * half
                cbase = cbases[pw]
                vals = [spat_v[pl.ds(sbase + v * _L, _L)] for v in range(vecs)]
                vals += [chrom_v[pl.ds(cbase + v * _L, _L)]
                         for v in range(vecs)]
                return vals

            def stores(pw, vals):
                for v in range(vecs):
                    plsc.addupdate(buf.at[pw, pl.ds(v * _L, _L)], vals[v])
                    plsc.addupdate(buf.at[pw, pl.ds(half + v * _L, _L)],
                                   vals[vecs + v])

            vals = loads(0)
            for pw in range(w - 1):
                nxt = loads(pw + 1)
                stores(pw, vals)
                vals = nxt
            stores(w - 1, vals)

        start_in(0, 0)
        start_in(1, 1)

        def ring_body(g, carry):
            for b in range(_NBUF):
                c = g * _NBUF + b
                wait_in(b)
                compute(c, b)
                start_out(c, b)

                @pl.when(c >= 2)
                def _():
                    wait_out((b - 2) % _NBUF)

                @pl.when(c + 2 < lines_per_w)
                def _():
                    start_in(c + 2, (b + 2) % _NBUF)
            return carry

        lax.fori_loop(0, lines_per_w // _NBUF, ring_body, 0, unroll=False)
        wait_out((lines_per_w - 2) % _NBUF)
        wait_out((lines_per_w - 1) % _NBUF)

    return body(x4, colors_flat, spat_flat, chrom_flat)


def kernel(x, color_indices, spatial_pe, chromatic_pe):
    b, h, w, d = x.shape
    half = d // 2
    colors_flat = color_indices.reshape(b * h * w).astype(jnp.int32)
    spat_flat = spatial_pe[:h, :w, :].reshape(h * w * half)
    chrom_flat = chromatic_pe.reshape(-1)
    return _sc_add_pe(x, colors_flat, spat_flat, chrom_flat, w, h, d)


# speedup vs baseline: 1.0068x; 1.0046x over previous
import jax
import jax.numpy as jnp
from jax import lax
from jax.experimental import pallas as pl
from jax.experimental.pallas import tpu as pltpu
from jax.experimental.pallas import tpu_sc as plsc

_NC = 2
_NS = 16
_NW = _NC * _NS
_L = 16

_NBUF = 4


def _sc_add_pe(x4, colors_flat, spat_flat, chrom_flat, w, hh, d):
    nb = x4.shape[0]
    n_lines = nb * hh
    half = d // 2
    vecs = half // _L
    lines_per_w = n_lines // _NW
    cols_per_w = lines_per_w * w

    mesh = plsc.VectorSubcoreMesh(core_axis_name="c", subcore_axis_name="s")

    @pl.kernel(
        out_type=jax.ShapeDtypeStruct((nb, hh, w, d), jnp.float32),
        mesh=mesh,
        compiler_params=pltpu.CompilerParams(
            needs_layout_passes=False, use_tc_tiling_on_sc=True),
        scratch_types=[
            pltpu.VMEM((hh * w * half,), jnp.float32),
            pltpu.VMEM((16 * half,), jnp.float32),
            pltpu.VMEM((cols_per_w + 16,), jnp.int32),
            pltpu.VMEM((w, d), jnp.float32),
            pltpu.VMEM((w, d), jnp.float32),
            pltpu.VMEM((w, d), jnp.float32),
            pltpu.VMEM((w, d), jnp.float32),
            pltpu.SemaphoreType.DMA,
            pltpu.SemaphoreType.DMA,
            pltpu.SemaphoreType.DMA,
            pltpu.SemaphoreType.DMA,
            pltpu.SemaphoreType.DMA,
            pltpu.SemaphoreType.DMA,
            pltpu.SemaphoreType.DMA,
            pltpu.SemaphoreType.DMA,
        ],
    )
    def body(x_hbm, col_hbm, spat_hbm, chrom_hbm, out_hbm,
             spat_v, chrom_v, col_v, rb0, rb1, rb2, rb3,
             si0, si1, si2, si3, so0, so1, so2, so3):
        wid = lax.axis_index("s") * _NC + lax.axis_index("c")
        base = wid * lines_per_w
        pltpu.sync_copy(spat_hbm, spat_v)
        pltpu.sync_copy(chrom_hbm, chrom_v.at[pl.ds(0, chrom_hbm.shape[0])])
        pltpu.sync_copy(col_hbm.at[pl.ds(wid * cols_per_w, cols_per_w)],
                        col_v.at[pl.ds(0, cols_per_w)])

        bufs = (rb0, rb1, rb2, rb3)
        isems = (si0, si1, si2, si3)
        osems = (so0, so1, so2, so3)

        def start_in(c, b):
            line = base + c
            pltpu.async_copy(x_hbm.at[lax.div(line, hh), lax.rem(line, hh)],
                             bufs[b], isems[b])

        def start_out(c, b):
            line = base + c
            pltpu.async_copy(bufs[b],
                             out_hbm.at[lax.div(line, hh), lax.rem(line, hh)],
                             osems[b])

        def wait_in(b):
            pltpu.make_async_copy(x_hbm.at[0, 0], bufs[b], isems[b]).wait()

        def wait_out(b):
            pltpu.make_async_copy(bufs[b], out_hbm.at[0, 0], osems[b]).wait()

        def compute(c, b):
            buf = bufs[b]
            line = base + c
            hbase = lax.rem(line, hh) * (w * half)
            coff = c * w
            col16a = col_v[pl.ds(coff, _L)]
            col16b = col_v[pl.ds(coff + _L, _L)]
            cbases = []
            for pw in range(w):
                crow = col16a[pw] if pw < _L else col16b[pw - _L]
                cbases.append(crow * half)
            for pw in range(w):
                sbase = hbase + pw * half
                cbase = cbases[pw]
                vals = [spat_v[pl.ds(sbase + v * _L, _L)] for v in range(vecs)]
                vals += [chrom_v[pl.ds(cbase + v * _L, _L)] for v in range(vecs)]
                for v in range(vecs):
                    plsc.addupdate(buf.at[pw, pl.ds(v * _L, _L)], vals[v])
                for v in range(vecs):
                    plsc.addupdate(buf.at[pw, pl.ds(half + v * _L, _L)],
                                   vals[vecs + v])

        start_in(0, 0)
        start_in(1, 1)

        def ring_body(g, carry):
            for b in range(_NBUF):
                c = g * _NBUF + b
                wait_in(b)
                compute(c, b)
                start_out(c, b)

                @pl.when(c >= 2)
                def _():
                    wait_out((b - 2) % _NBUF)

                @pl.when(c + 2 < lines_per_w)
                def _():
                    start_in(c + 2, (b + 2) % _NBUF)
            return carry

        lax.fori_loop(0, lines_per_w // _NBUF, ring_body, 0, unroll=False)
        wait_out((lines_per_w - 2) % _NBUF)
        wait_out((lines_per_w - 1) % _NBUF)

    return body(x4, colors_flat, spat_flat, chrom_flat)


def kernel(x, color_indices, spatial_pe, chromatic_pe):
    b, h, w, d = x.shape
    half = d // 2
    colors_flat = color_indices.reshape(b * h * w).astype(jnp.int32)
    spat_flat = spatial_pe[:h, :w, :].reshape(h * w * half)
    chrom_flat = chromatic_pe.reshape(-1)
    return _sc_add_pe(x, colors_flat, spat_flat, chrom_flat, w, h, d)
